# baseline (speedup 1.0000x reference)
import functools

import numpy as np

import jax
import jax.numpy as jnp
from jax import lax
from jax.experimental import pallas as pl
from jax.experimental.pallas import tpu as pltpu
from jax.experimental.pallas import tpu_sc as plsc

B = 4096
D = 128
N_TRIES = 150
NT_PAD = 256

ROW_TILE = 512
COL_TILE = 512
NC, NS, L = 2, 16, 16
NW = NC * NS
B_PER_W = B // NW
GROUPS = B_PER_W // L

_BIG_I = 2**30
_BIG_F = 3e38


def _tc_body(lim_ref, hd_ref, hdt_ref, sqr_ref, sqc_ref, tgtc_ref, tgt32_ref,
             tgt2_ref, up_ref, un_ref, dst_ref, pos_ref, nc_ref):
    i = pl.program_id(0)
    j = pl.program_id(1)

    @pl.when(j <= lim_ref[0])
    def _():
        a = hd_ref[...]
        mm = lax.dot_general(a, hdt_ref[...], (((1,), (0,)), ((), ())))
        dst_ref[...] = (mm + sqr_ref[...]) + sqc_ref[...]

    @pl.when((i == 0) & (j == 0))
    def _():
        tgtc = tgtc_ref[...]
        n_sp = jnp.sum(tgtc)
        n_ge = B - n_sp
        tgt32 = tgt32_ref[...]
        pool32 = jnp.where(tgt32 == 0, n_ge, n_sp)
        idx32 = (lax.broadcasted_iota(jnp.int32, (B // D, D), 0) * D
                 + lax.broadcasted_iota(jnp.int32, (B // D, D), 1))
        up = up_ref[...]
        pos = jnp.minimum((up * pool32.astype(jnp.float32)).astype(jnp.int32),
                          pool32 - 1)
        pos = jnp.where(pos == idx32, (pos + 1) % pool32, pos)
        pos_ref[...] = pos
        neg_pool = jnp.where(tgt2_ref[...] == 0, n_sp, n_ge)
        un = un_ref[...]
        nc = jnp.minimum((un * neg_pool.astype(jnp.float32)).astype(jnp.int32),
                         neg_pool - 1)
        nc_ref[...] = nc


def _tc_stage(lim, hd, hdt, sqr, sqc, tgtc, tgt32, tgt2, up, un):
    def dstmap(i, j, s):
        return (i, jnp.minimum(j, s[0]))

    return pl.pallas_call(
        _tc_body,
        grid_spec=pltpu.PrefetchScalarGridSpec(
            num_scalar_prefetch=1,
            grid=(B // ROW_TILE, B // COL_TILE),
            in_specs=[
                pl.BlockSpec((ROW_TILE, D), lambda i, j, s: (i, 0)),
                pl.BlockSpec((D, COL_TILE),
                             lambda i, j, s: (0, jnp.minimum(j, s[0]))),
                pl.BlockSpec((ROW_TILE, 1), lambda i, j, s: (i, 0)),
                pl.BlockSpec((1, COL_TILE),
                             lambda i, j, s: (0, jnp.minimum(j, s[0]))),
                pl.BlockSpec((B, 1), lambda i, j, s: (0, 0)),
                pl.BlockSpec((B // D, D), lambda i, j, s: (0, 0)),
                pl.BlockSpec((2 * B, 1), lambda i, j, s: (0, 0)),
                pl.BlockSpec((B // D, D), lambda i, j, s: (0, 0)),
                pl.BlockSpec((2 * B, D), lambda i, j, s: (0, 0)),
            ],
            out_specs=[
                pl.BlockSpec((ROW_TILE, COL_TILE), dstmap),
                pl.BlockSpec((B // D, D), lambda i, j, s: (0, 0)),
                pl.BlockSpec((2 * B, D), lambda i, j, s: (0, 0)),
            ],
        ),
        out_shape=[
            jax.ShapeDtypeStruct((B, B), jnp.float32),
            jax.ShapeDtypeStruct((B // D, D), jnp.int32),
            jax.ShapeDtypeStruct((2 * B, D), jnp.int32),
        ],
    )(lim, hd, hdt, sqr, sqc, tgtc, tgt32, tgt2, up, un)


def _sc_body(dst_hbm, hd_hbm, pos_hbm, nc_hbm, marg_hbm, out_hbm,
             dstbuf, ncbuf, posbuf, margbuf, idxb, outrows, sem):
    wid = lax.axis_index("s") * NC + lax.axis_index("c")
    base = wid * B_PER_W
    iota16 = lax.iota(jnp.int32, L)
    pltpu.sync_copy(marg_hbm, margbuf)
    pltpu.sync_copy(pos_hbm.at[wid], posbuf)
    mvec = margbuf[...]

    for g in range(GROUPS):
        rowbase = base + g * L
        pltpu.sync_copy(dst_hbm.at[pl.ds(rowbase, L)], dstbuf)
        pltpu.sync_copy(nc_hbm.at[pl.ds(2 * rowbase, 2 * L), :], ncbuf)
        posv = posbuf[pl.ds(g * L, L)]
        dp = plsc.load_gather(dstbuf, [iota16, posv >> 7, posv & 127])

        def step(j, carry):
            fs, fc, dmin, mc = carry
            jv = jnp.broadcast_to(j, (L,)).astype(jnp.int32)
            cand = plsc.load_gather(ncbuf, [2 * iota16 + (jv >> 7), jv & 127])
            dj = plsc.load_gather(dstbuf, [iota16, cand >> 7, cand & 127])
            semi = (dp < dj) & ((dp - dj + mvec) > 0)
            newf = semi & (fs >= _BIG_I)
            fs = jnp.where(newf, jv, fs)
            fc = jnp.where(newf, cand, fc)
            upd = dj < dmin
            dmin = jnp.where(upd, dj, dmin)
            mc = jnp.where(upd, cand, mc)
            return fs, fc, dmin, mc

        init = (jnp.full((L,), _BIG_I, jnp.int32), jnp.zeros((L,), jnp.int32),
                jnp.full((L,), _BIG_F, jnp.float32), jnp.zeros((L,), jnp.int32))
        fs, fc, dmin, mc = lax.fori_loop(0, N_TRIES, step, init)
        negv = jnp.where(fs < _BIG_I, fc, mc)

        la = g * L + iota16
        plsc.store_scatter(idxb, [jnp.zeros((L,), jnp.int32), la], posv)
        plsc.store_scatter(idxb, [jnp.ones((L,), jnp.int32), la], negv)

    pltpu.sync_copy(hd_hbm.at[pl.ds(base, B_PER_W)], outrows)
    pltpu.sync_copy(outrows, out_hbm.at[pl.ds(base, B_PER_W)])
    for k in range(2):
        pltpu.async_copy(hd_hbm.at[idxb.at[k]], outrows, sem).wait()
        pltpu.sync_copy(outrows,
                        out_hbm.at[pl.ds((k + 1) * B + base, B_PER_W)])


@functools.cache
def _make_sc_stage():
    return pl.kernel(
        _sc_body,
        out_type=jax.ShapeDtypeStruct((3 * B, D), jnp.float32),
        mesh=plsc.VectorSubcoreMesh(core_axis_name="c", subcore_axis_name="s",
                                    num_cores=NC, num_subcores=NS),
        scratch_types=[
            pltpu.VMEM((L, B // D, D), jnp.float32),
            pltpu.VMEM((2 * L, D), jnp.int32),
            pltpu.VMEM((B_PER_W,), jnp.int32),
            pltpu.VMEM((L,), jnp.float32),
            pltpu.VMEM((2, B_PER_W), jnp.int32),
            pltpu.VMEM((B_PER_W, D), jnp.float32),
            pltpu.SemaphoreType.DMA,
        ],
        compiler_params=pltpu.CompilerParams(use_tc_tiling_on_sc=False,
                                             needs_layout_passes=False),
    )


def _threefry2x32(key, x0, x1):
    rot1 = (13, 15, 26, 6)
    rot2 = (17, 29, 16, 24)
    with np.errstate(over="ignore"):
        ks0, ks1 = np.uint32(key[0]), np.uint32(key[1])
        ks2 = np.uint32(ks0 ^ ks1 ^ np.uint32(0x1BD11BDA))
        x0 = (x0 + ks0).astype(np.uint32)
        x1 = (x1 + ks1).astype(np.uint32)
        ks = (ks0, ks1, ks2)
        for i in range(5):
            for r in rot1 if i % 2 == 0 else rot2:
                x0 = (x0 + x1).astype(np.uint32)
                x1 = ((x1 << np.uint32(r)) | (x1 >> np.uint32(32 - r))).astype(
                    np.uint32)
                x1 = (x1 ^ x0).astype(np.uint32)
            x0 = (x0 + ks[(i + 1) % 3]).astype(np.uint32)
            x1 = (x1 + ks[(i + 2) % 3] + np.uint32(i + 1)).astype(np.uint32)
    return x0, x1


def _random_bits(key, n, partitionable):
    if partitionable:
        o0, o1 = _threefry2x32(key, np.zeros(n, np.uint32),
                               np.arange(n, dtype=np.uint32))
        return o0 ^ o1
    m = n + n % 2
    cnt = np.arange(m, dtype=np.uint32)
    o0, o1 = _threefry2x32(key, cnt[: m // 2], cnt[m // 2:])
    return np.concatenate([o0, o1])[:n]


def _np_uniform(key, shape, partitionable):
    bits = _random_bits(key, int(np.prod(shape)), partitionable)
    fl = ((bits >> np.uint32(9)) | np.uint32(0x3F800000)).view(np.float32) - 1.0
    return np.maximum(np.float32(0.0), fl).reshape(shape)


@functools.cache
def _fixed_uniforms():
    part = bool(jax.config.jax_threefry_partitionable)
    key = (np.uint32(0), np.uint32(1234))
    if part:
        o0, o1 = _threefry2x32(key, np.zeros(2, np.uint32),
                               np.arange(2, dtype=np.uint32))
        kp, kn = (o0[0], o1[0]), (o0[1], o1[1])
    else:
        bits = _random_bits(key, 4, part)
        kp, kn = (bits[0], bits[1]), (bits[2], bits[3])
    up = _np_uniform(kp, (B,), part).reshape(B // D, D)
    un = np.concatenate(
        [_np_uniform(kn, (B, N_TRIES), part),
         np.zeros((B, NT_PAD - N_TRIES), np.float32)], axis=1)
    return up, un.reshape(2 * B, D)


def kernel(batch_hd, batch_tgt, margin):
    hd = batch_hd.astype(jnp.float32)
    sq = jnp.sum(hd * hd, axis=-1)
    up, un = _fixed_uniforms()

    tgt = batch_tgt.astype(jnp.int32)
    tgt2 = jnp.repeat(tgt, 2).reshape(2 * B, 1)
    n_sp = jnp.sum(tgt)
    maxpool = jnp.maximum(n_sp, B - n_sp)
    lim = ((maxpool + COL_TILE - 1) // COL_TILE - 1).reshape(1)
    dst, pos, nc = _tc_stage(
        lim, -2.0 * hd, hd.T, sq.reshape(B, 1), sq.reshape(1, B),
        tgt.reshape(B, 1), tgt.reshape(B // D, D), tgt2, up, un)

    marg = jnp.full((L,), margin, jnp.float32)
    out = _make_sc_stage()(dst.reshape(B, B // D, D), hd, pos, nc, marg)
    return out.reshape(3, B, D).transpose(1, 0, 2)


# speedup vs baseline: 1.3941x; 1.3941x over previous
import functools

import numpy as np

import jax
import jax.numpy as jnp
from jax import lax
from jax.experimental import pallas as pl
from jax.experimental.pallas import tpu as pltpu
from jax.experimental.pallas import tpu_sc as plsc

B = 4096
D = 128
N_TRIES = 150
NT_PAD = 256

ROW_TILE = 512
COL_TILE = 1024
NC, NS, L = 2, 16, 16
NW = NC * NS
B_PER_W = B // NW
GROUPS = B_PER_W // L

_BIG_I = 2**30
_BIG_F = 3e38


def _tc_body(lim_ref, hd_ref, hdt_ref, sqr_ref, sqc_ref, tgtc_ref, tgt32_ref,
             tgt2_ref, up_ref, un_ref, dst_ref, pos_ref, nc_ref):
    i = pl.program_id(0)
    j = pl.program_id(1)

    @pl.when(j <= lim_ref[0])
    def _():
        a = hd_ref[...]
        mm = lax.dot_general(a, hdt_ref[...], (((1,), (0,)), ((), ())))
        dst_ref[...] = ((mm + sqr_ref[...]) + sqc_ref[...]).reshape(
            ROW_TILE, COL_TILE // D, D)

    @pl.when((i == 0) & (j == 0))
    def _():
        tgtc = tgtc_ref[...]
        n_sp = jnp.sum(tgtc)
        n_ge = B - n_sp
        tgt32 = tgt32_ref[...]
        pool32 = jnp.where(tgt32 == 0, n_ge, n_sp)
        idx32 = (lax.broadcasted_iota(jnp.int32, (B // D, D), 0) * D
                 + lax.broadcasted_iota(jnp.int32, (B // D, D), 1))
        up = up_ref[...]
        pos = jnp.minimum((up * pool32.astype(jnp.float32)).astype(jnp.int32),
                          pool32 - 1)
        pos = jnp.where(pos == idx32, (pos + 1) % pool32, pos)
        pos_ref[...] = pos
        neg_pool = jnp.where(tgt2_ref[...] == 0, n_sp, n_ge)
        un = un_ref[...]
        nc = jnp.minimum((un * neg_pool.astype(jnp.float32)).astype(jnp.int32),
                         neg_pool - 1)
        nc_ref[...] = nc


def _tc_stage(lim, hd, hdt, sqr, sqc, tgtc, tgt32, tgt2, up, un):
    def dstmap(i, j, s):
        return (i, jnp.minimum(j, s[0]), 0)

    return pl.pallas_call(
        _tc_body,
        grid_spec=pltpu.PrefetchScalarGridSpec(
            num_scalar_prefetch=1,
            grid=(B // ROW_TILE, B // COL_TILE),
            in_specs=[
                pl.BlockSpec((ROW_TILE, D), lambda i, j, s: (i, 0)),
                pl.BlockSpec((D, COL_TILE),
                             lambda i, j, s: (0, jnp.minimum(j, s[0]))),
                pl.BlockSpec((ROW_TILE, 1), lambda i, j, s: (i, 0)),
                pl.BlockSpec((1, COL_TILE),
                             lambda i, j, s: (0, jnp.minimum(j, s[0]))),
                pl.BlockSpec((B, 1), lambda i, j, s: (0, 0)),
                pl.BlockSpec((B // D, D), lambda i, j, s: (0, 0)),
                pl.BlockSpec((2 * B, 1), lambda i, j, s: (0, 0)),
                pl.BlockSpec((B // D, D), lambda i, j, s: (0, 0)),
                pl.BlockSpec((2 * B, D), lambda i, j, s: (0, 0)),
            ],
            out_specs=[
                pl.BlockSpec((ROW_TILE, COL_TILE // D, D), dstmap),
                pl.BlockSpec((B // D, D), lambda i, j, s: (0, 0)),
                pl.BlockSpec((2 * B, D), lambda i, j, s: (0, 0)),
            ],
        ),
        out_shape=[
            jax.ShapeDtypeStruct((B, B // D, D), jnp.float32),
            jax.ShapeDtypeStruct((B // D, D), jnp.int32),
            jax.ShapeDtypeStruct((2 * B, D), jnp.int32),
        ],
    )(lim, hd, hdt, sqr, sqc, tgtc, tgt32, tgt2, up, un)


def _sc_body(dst_hbm, hd_hbm, pos_hbm, nc_hbm, marg_hbm, out_hbm,
             dstbuf, ncbuf, posbuf, margbuf, idxb, outrows, sem):
    wid = lax.axis_index("s") * NC + lax.axis_index("c")
    base = wid * B_PER_W
    iota16 = lax.iota(jnp.int32, L)
    pltpu.sync_copy(marg_hbm, margbuf)
    pltpu.sync_copy(pos_hbm.at[wid], posbuf)
    mvec = margbuf[...]

    for g in range(GROUPS):
        rowbase = base + g * L
        pltpu.sync_copy(dst_hbm.at[pl.ds(rowbase, L)], dstbuf)
        pltpu.sync_copy(nc_hbm.at[pl.ds(2 * rowbase, 2 * L), :], ncbuf)
        posv = posbuf[pl.ds(g * L, L)]
        dp = plsc.load_gather(dstbuf, [iota16, posv >> 7, posv & 127])

        def step(j, carry):
            fs, fc, dmin, mc = carry
            jv = jnp.broadcast_to(j, (L,)).astype(jnp.int32)
            cand = plsc.load_gather(ncbuf, [2 * iota16 + (jv >> 7), jv & 127])
            dj = plsc.load_gather(dstbuf, [iota16, cand >> 7, cand & 127])
            semi = (dp < dj) & ((dp - dj + mvec) > 0)
            newf = semi & (fs >= _BIG_I)
            fs = jnp.where(newf, jv, fs)
            fc = jnp.where(newf, cand, fc)
            upd = dj < dmin
            dmin = jnp.where(upd, dj, dmin)
            mc = jnp.where(upd, cand, mc)
            return fs, fc, dmin, mc

        init = (jnp.full((L,), _BIG_I, jnp.int32), jnp.zeros((L,), jnp.int32),
                jnp.full((L,), _BIG_F, jnp.float32), jnp.zeros((L,), jnp.int32))
        fs, fc, dmin, mc = lax.fori_loop(0, N_TRIES, step, init)
        negv = jnp.where(fs < _BIG_I, fc, mc)

        la = g * L + iota16
        plsc.store_scatter(idxb, [jnp.zeros((L,), jnp.int32), la], posv)
        plsc.store_scatter(idxb, [jnp.ones((L,), jnp.int32), la], negv)

    pltpu.sync_copy(hd_hbm.at[pl.ds(base, B_PER_W)], outrows)
    pltpu.sync_copy(outrows, out_hbm.at[pl.ds(base, B_PER_W)])
    for k in range(2):
        pltpu.async_copy(hd_hbm.at[idxb.at[k]], outrows, sem).wait()
        pltpu.sync_copy(outrows,
                        out_hbm.at[pl.ds((k + 1) * B + base, B_PER_W)])


@functools.cache
def _make_sc_stage():
    return pl.kernel(
        _sc_body,
        out_type=jax.ShapeDtypeStruct((3 * B, D), jnp.float32),
        mesh=plsc.VectorSubcoreMesh(core_axis_name="c", subcore_axis_name="s",
                                    num_cores=NC, num_subcores=NS),
        scratch_types=[
            pltpu.VMEM((L, B // D, D), jnp.float32),
            pltpu.VMEM((2 * L, D), jnp.int32),
            pltpu.VMEM((B_PER_W,), jnp.int32),
            pltpu.VMEM((L,), jnp.float32),
            pltpu.VMEM((2, B_PER_W), jnp.int32),
            pltpu.VMEM((B_PER_W, D), jnp.float32),
            pltpu.SemaphoreType.DMA,
        ],
        compiler_params=pltpu.CompilerParams(use_tc_tiling_on_sc=False,
                                             needs_layout_passes=False),
    )


def _threefry2x32(key, x0, x1):
    rot1 = (13, 15, 26, 6)
    rot2 = (17, 29, 16, 24)
    with np.errstate(over="ignore"):
        ks0, ks1 = np.uint32(key[0]), np.uint32(key[1])
        ks2 = np.uint32(ks0 ^ ks1 ^ np.uint32(0x1BD11BDA))
        x0 = (x0 + ks0).astype(np.uint32)
        x1 = (x1 + ks1).astype(np.uint32)
        ks = (ks0, ks1, ks2)
        for i in range(5):
            for r in rot1 if i % 2 == 0 else rot2:
                x0 = (x0 + x1).astype(np.uint32)
                x1 = ((x1 << np.uint32(r)) | (x1 >> np.uint32(32 - r))).astype(
                    np.uint32)
                x1 = (x1 ^ x0).astype(np.uint32)
            x0 = (x0 + ks[(i + 1) % 3]).astype(np.uint32)
            x1 = (x1 + ks[(i + 2) % 3] + np.uint32(i + 1)).astype(np.uint32)
    return x0, x1


def _random_bits(key, n, partitionable):
    if partitionable:
        o0, o1 = _threefry2x32(key, np.zeros(n, np.uint32),
                               np.arange(n, dtype=np.uint32))
        return o0 ^ o1
    m = n + n % 2
    cnt = np.arange(m, dtype=np.uint32)
    o0, o1 = _threefry2x32(key, cnt[: m // 2], cnt[m // 2:])
    return np.concatenate([o0, o1])[:n]


def _np_uniform(key, shape, partitionable):
    bits = _random_bits(key, int(np.prod(shape)), partitionable)
    fl = ((bits >> np.uint32(9)) | np.uint32(0x3F800000)).view(np.float32) - 1.0
    return np.maximum(np.float32(0.0), fl).reshape(shape)


@functools.cache
def _fixed_uniforms():
    part = bool(jax.config.jax_threefry_partitionable)
    key = (np.uint32(0), np.uint32(1234))
    if part:
        o0, o1 = _threefry2x32(key, np.zeros(2, np.uint32),
                               np.arange(2, dtype=np.uint32))
        kp, kn = (o0[0], o1[0]), (o0[1], o1[1])
    else:
        bits = _random_bits(key, 4, part)
        kp, kn = (bits[0], bits[1]), (bits[2], bits[3])
    up = _np_uniform(kp, (B,), part).reshape(B // D, D)
    un = np.concatenate(
        [_np_uniform(kn, (B, N_TRIES), part),
         np.zeros((B, NT_PAD - N_TRIES), np.float32)], axis=1)
    return up, un.reshape(2 * B, D)


def kernel(batch_hd, batch_tgt, margin):
    hd = batch_hd.astype(jnp.float32)
    sq = jnp.sum(hd * hd, axis=-1)
    up, un = _fixed_uniforms()

    tgt = batch_tgt.astype(jnp.int32)
    tgt2 = jnp.repeat(tgt, 2).reshape(2 * B, 1)
    n_sp = jnp.sum(tgt)
    maxpool = jnp.maximum(n_sp, B - n_sp)
    lim = ((maxpool + COL_TILE - 1) // COL_TILE - 1).reshape(1)
    dst, pos, nc = _tc_stage(
        lim, -2.0 * hd, hd.T, sq.reshape(B, 1), sq.reshape(1, B),
        tgt.reshape(B, 1), tgt.reshape(B // D, D), tgt2, up, un)

    marg = jnp.full((L,), margin, jnp.float32)
    out = _make_sc_stage()(dst, hd, pos, nc, marg)
    return out.reshape(3, B, D).transpose(1, 0, 2)


# speedup vs baseline: 1.8563x; 1.3315x over previous
import functools

import numpy as np

import jax
import jax.numpy as jnp
from jax import lax
from jax.experimental import pallas as pl
from jax.experimental.pallas import tpu as pltpu
from jax.experimental.pallas import tpu_sc as plsc

B = 4096
D = 128
N_TRIES = 150
NT_PAD = 256

ROW_TILE = 512
NC, NS, L = 2, 16, 16
NW = NC * NS
B_PER_W = B // NW
GROUPS = B_PER_W // L
SLIM_CH = 20
NCHUNK = B // D

_BIG_I = 2**30
_BIG_F = 3e38


def _tc_body(hd_ref, hdt_ref, sqr_ref, sqc_ref, tgtc_ref, tgt32_ref, tgt2_ref,
             up_ref, un_ref, dst_ref, pos_ref, nc_ref):
    i = pl.program_id(0)
    a = hd_ref[...]
    mm = lax.dot_general(a, hdt_ref[...], (((1,), (0,)), ((), ())))
    dst_ref[...] = ((mm + sqr_ref[...]) + sqc_ref[...]).reshape(
        ROW_TILE, NCHUNK, D)

    @pl.when(i == 0)
    def _():
        tgtc = tgtc_ref[...]
        n_sp = jnp.sum(tgtc)
        n_ge = B - n_sp
        tgt32 = tgt32_ref[...]
        pool32 = jnp.where(tgt32 == 0, n_ge, n_sp)
        idx32 = (lax.broadcasted_iota(jnp.int32, (B // D, D), 0) * D
                 + lax.broadcasted_iota(jnp.int32, (B // D, D), 1))
        up = up_ref[...]
        pos = jnp.minimum((up * pool32.astype(jnp.float32)).astype(jnp.int32),
                          pool32 - 1)
        pos = jnp.where(pos == idx32, (pos + 1) % pool32, pos)
        pos_ref[...] = pos
        neg_pool = jnp.where(tgt2_ref[...] == 0, n_sp, n_ge)
        un = un_ref[...]
        nc = jnp.minimum((un * neg_pool.astype(jnp.float32)).astype(jnp.int32),
                         neg_pool - 1)
        nc_ref[...] = nc


def _tc_stage(hd, hdt, sqr, sqc, tgtc, tgt32, tgt2, up, un):
    return pl.pallas_call(
        _tc_body,
        grid=(B // ROW_TILE,),
        in_specs=[
            pl.BlockSpec((ROW_TILE, D), lambda i: (i, 0)),
            pl.BlockSpec((D, B), lambda i: (0, 0)),
            pl.BlockSpec((ROW_TILE, 1), lambda i: (i, 0)),
            pl.BlockSpec((1, B), lambda i: (0, 0)),
            pl.BlockSpec((B, 1), lambda i: (0, 0)),
            pl.BlockSpec((B // D, D), lambda i: (0, 0)),
            pl.BlockSpec((2 * B, 1), lambda i: (0, 0)),
            pl.BlockSpec((B // D, D), lambda i: (0, 0)),
            pl.BlockSpec((2 * B, D), lambda i: (0, 0)),
        ],
        out_specs=[
            pl.BlockSpec((ROW_TILE, NCHUNK, D), lambda i: (i, 0, 0)),
            pl.BlockSpec((B // D, D), lambda i: (0, 0)),
            pl.BlockSpec((2 * B, D), lambda i: (0, 0)),
        ],
        out_shape=[
            jax.ShapeDtypeStruct((B, NCHUNK, D), jnp.float32),
            jax.ShapeDtypeStruct((B // D, D), jnp.int32),
            jax.ShapeDtypeStruct((2 * B, D), jnp.int32),
        ],
    )(hd, hdt, sqr, sqc, tgtc, tgt32, tgt2, up, un)


def _select_group(g, iota16, mvec, ncbuf, posbuf, idxb, gather_dj):
    posv = posbuf[pl.ds(g * L, L)]
    dp = gather_dj(posv)

    def step(j, carry):
        fs, fc, dmin, mc = carry
        jv = jnp.broadcast_to(j, (L,)).astype(jnp.int32)
        cand = plsc.load_gather(ncbuf, [2 * iota16 + (jv >> 7), jv & 127])
        dj = gather_dj(cand)
        semi = (dp < dj) & ((dp - dj + mvec) > 0)
        newf = semi & (fs >= _BIG_I)
        fs = jnp.where(newf, jv, fs)
        fc = jnp.where(newf, cand, fc)
        upd = dj < dmin
        dmin = jnp.where(upd, dj, dmin)
        mc = jnp.where(upd, cand, mc)
        return fs, fc, dmin, mc

    init = (jnp.full((L,), _BIG_I, jnp.int32), jnp.zeros((L,), jnp.int32),
            jnp.full((L,), _BIG_F, jnp.float32), jnp.zeros((L,), jnp.int32))
    fs, fc, dmin, mc = lax.fori_loop(0, N_TRIES, step, init)
    negv = jnp.where(fs < _BIG_I, fc, mc)

    la = g * L + iota16
    plsc.store_scatter(idxb, [jnp.zeros((L,), jnp.int32), la], posv)
    plsc.store_scatter(idxb, [jnp.ones((L,), jnp.int32), la], negv)


def _sc_body(dst_hbm, hd_hbm, pos_hbm, nc_hbm, marg_hbm, out_hbm,
             buf0, buf1, ncbuf, posbuf, margbuf, idxb, outrows,
             sem, sem0, sem1):
    wid = lax.axis_index("s") * NC + lax.axis_index("c")
    base = wid * B_PER_W
    iota16 = lax.iota(jnp.int32, L)
    pltpu.sync_copy(marg_hbm, margbuf)
    pltpu.sync_copy(pos_hbm.at[wid], posbuf)

    mvec = plsc.load_gather(margbuf, [jnp.zeros((L,), jnp.int32)])
    mpv = plsc.load_gather(margbuf, [jnp.full((L,), 8, jnp.int32)])
    mp = jnp.max(mpv.astype(jnp.int32))

    bufs = (buf0, buf1)
    sems = (sem0, sem1)

    def slim_path():
        def start(g):
            rowbase = base + g * L
            return pltpu.async_copy(
                dst_hbm.at[pl.ds(rowbase, L), pl.ds(0, SLIM_CH)],
                bufs[g % 2], sems[g % 2])

        handles = [start(0)]
        for g in range(GROUPS):
            rowbase = base + g * L
            pltpu.sync_copy(nc_hbm.at[pl.ds(2 * rowbase, 2 * L), :], ncbuf)
            handles[g].wait()
            if g + 1 < GROUPS:
                handles.append(start(g + 1))
            buf = bufs[g % 2]

            def gather_dj(cand, buf=buf):
                return plsc.load_gather(buf, [iota16, cand >> 7, cand & 127])

            _select_group(g, iota16, mvec, ncbuf, posbuf, idxb, gather_dj)

    def full_path():
        for g in range(GROUPS):
            rowbase = base + g * L
            pltpu.sync_copy(nc_hbm.at[pl.ds(2 * rowbase, 2 * L), :], ncbuf)
            pltpu.sync_copy(dst_hbm.at[pl.ds(rowbase, L), pl.ds(0, L)],
                            buf0.at[:, pl.ds(0, L)])
            pltpu.sync_copy(dst_hbm.at[pl.ds(rowbase, L), pl.ds(L, L)],
                            buf1.at[:, pl.ds(0, L)])

            def gather_dj(cand):
                ch = cand >> 7
                cm = ch & (L - 1)
                cl = cand & 127
                d0 = plsc.load_gather(buf0, [iota16, cm, cl])
                d1 = plsc.load_gather(buf1, [iota16, cm, cl])
                return jnp.where(ch < L, d0, d1)

            _select_group(g, iota16, mvec, ncbuf, posbuf, idxb, gather_dj)

    lax.cond(mp <= SLIM_CH * D, slim_path, full_path)

    pltpu.sync_copy(hd_hbm.at[pl.ds(base, B_PER_W)], outrows)
    pltpu.sync_copy(outrows, out_hbm.at[pl.ds(base, B_PER_W)])
    for k in range(2):
        pltpu.async_copy(hd_hbm.at[idxb.at[k]], outrows, sem).wait()
        pltpu.sync_copy(outrows,
                        out_hbm.at[pl.ds((k + 1) * B + base, B_PER_W)])


@functools.cache
def _make_sc_stage():
    return pl.kernel(
        _sc_body,
        out_type=jax.ShapeDtypeStruct((3 * B, D), jnp.float32),
        mesh=plsc.VectorSubcoreMesh(core_axis_name="c", subcore_axis_name="s",
                                    num_cores=NC, num_subcores=NS),
        scratch_types=[
            pltpu.VMEM((L, SLIM_CH, D), jnp.float32),
            pltpu.VMEM((L, SLIM_CH, D), jnp.float32),
            pltpu.VMEM((2 * L, D), jnp.int32),
            pltpu.VMEM((B_PER_W,), jnp.int32),
            pltpu.VMEM((L,), jnp.float32),
            pltpu.VMEM((2, B_PER_W), jnp.int32),
            pltpu.VMEM((B_PER_W, D), jnp.float32),
            pltpu.SemaphoreType.DMA,
            pltpu.SemaphoreType.DMA,
            pltpu.SemaphoreType.DMA,
        ],
        compiler_params=pltpu.CompilerParams(use_tc_tiling_on_sc=False,
                                             needs_layout_passes=False),
    )


def _threefry2x32(key, x0, x1):
    rot1 = (13, 15, 26, 6)
    rot2 = (17, 29, 16, 24)
    with np.errstate(over="ignore"):
        ks0, ks1 = np.uint32(key[0]), np.uint32(key[1])
        ks2 = np.uint32(ks0 ^ ks1 ^ np.uint32(0x1BD11BDA))
        x0 = (x0 + ks0).astype(np.uint32)
        x1 = (x1 + ks1).astype(np.uint32)
        ks = (ks0, ks1, ks2)
        for i in range(5):
            for r in rot1 if i % 2 == 0 else rot2:
                x0 = (x0 + x1).astype(np.uint32)
                x1 = ((x1 << np.uint32(r)) | (x1 >> np.uint32(32 - r))).astype(
                    np.uint32)
                x1 = (x1 ^ x0).astype(np.uint32)
            x0 = (x0 + ks[(i + 1) % 3]).astype(np.uint32)
            x1 = (x1 + ks[(i + 2) % 3] + np.uint32(i + 1)).astype(np.uint32)
    return x0, x1


def _random_bits(key, n, partitionable):
    if partitionable:
        o0, o1 = _threefry2x32(key, np.zeros(n, np.uint32),
                               np.arange(n, dtype=np.uint32))
        return o0 ^ o1
    m = n + n % 2
    cnt = np.arange(m, dtype=np.uint32)
    o0, o1 = _threefry2x32(key, cnt[: m // 2], cnt[m // 2:])
    return np.concatenate([o0, o1])[:n]


def _np_uniform(key, shape, partitionable):
    bits = _random_bits(key, int(np.prod(shape)), partitionable)
    fl = ((bits >> np.uint32(9)) | np.uint32(0x3F800000)).view(np.float32) - 1.0
    return np.maximum(np.float32(0.0), fl).reshape(shape)


@functools.cache
def _fixed_uniforms():
    part = bool(jax.config.jax_threefry_partitionable)
    key = (np.uint32(0), np.uint32(1234))
    if part:
        o0, o1 = _threefry2x32(key, np.zeros(2, np.uint32),
                               np.arange(2, dtype=np.uint32))
        kp, kn = (o0[0], o1[0]), (o0[1], o1[1])
    else:
        bits = _random_bits(key, 4, part)
        kp, kn = (bits[0], bits[1]), (bits[2], bits[3])
    up = _np_uniform(kp, (B,), part).reshape(B // D, D)
    un = np.concatenate(
        [_np_uniform(kn, (B, N_TRIES), part),
         np.zeros((B, NT_PAD - N_TRIES), np.float32)], axis=1)
    return up, un.reshape(2 * B, D)


def kernel(batch_hd, batch_tgt, margin):
    hd = batch_hd.astype(jnp.float32)
    sq = jnp.sum(hd * hd, axis=-1)
    up, un = _fixed_uniforms()

    tgt = batch_tgt.astype(jnp.int32)
    tgt2 = jnp.repeat(tgt, 2).reshape(2 * B, 1)
    dst, pos, nc = _tc_stage(
        -2.0 * hd, hd.T, sq.reshape(B, 1), sq.reshape(1, B),
        tgt.reshape(B, 1), tgt.reshape(B // D, D), tgt2, up, un)

    n_sp = jnp.sum(tgt)
    maxpool = jnp.maximum(n_sp, B - n_sp)
    marg = jnp.where(jnp.arange(L) < 8, jnp.asarray(margin, jnp.float32),
                     maxpool.astype(jnp.float32))
    out = _make_sc_stage()(dst, hd, pos, nc, marg)
    return out.reshape(3, B, D).transpose(1, 0, 2)
